# Initial kernel scaffold; baseline (speedup 1.0000x reference)
import jax
import jax.numpy as jnp
from jax.experimental import pallas as pl


def kernel(x, edge_index, W1, b1, W2, b2, W4, b4):
    raise NotImplementedError("write your pallas kernel here")


# speedup vs baseline: 6.7670x; 6.7670x over previous
import functools

import jax
import jax.numpy as jnp
from jax import lax
from jax.experimental import pallas as pl
from jax.experimental.pallas import tpu as pltpu
from jax.experimental.pallas import tpu_sc as plsc

N = 10000
D = 128
NC = 2
NS = 16
NW = NC * NS
K = 64
CH = 160
EPW = CH * K
EPAD = NW * EPW
NPAD = 10240
RPT = NPAD // NS
F32 = jnp.float32

_mesh = plsc.VectorSubcoreMesh(core_axis_name="c", subcore_axis_name="s")
_sc_params = pltpu.CompilerParams(needs_layout_passes=False)


@functools.partial(
    pl.kernel,
    out_type=[jax.ShapeDtypeStruct((NW, NPAD), F32),
              jax.ShapeDtypeStruct((NW, NPAD), F32)],
    mesh=_mesh,
    compiler_params=_sc_params,
    scratch_types=[pltpu.VMEM((EPW,), jnp.int32),
                   pltpu.VMEM((NPAD,), F32),
                   pltpu.VMEM((NPAD,), F32)],
)
def _sc_degrees(src_hbm, dst_hbm, dout_hbm, din_hbm, idxv, dov, div):
    cid = lax.axis_index("c")
    sid = lax.axis_index("s")
    wid = sid * NC + cid
    zeros = jnp.zeros((16,), F32)
    ones = jnp.ones((16,), F32)

    def zbody(i, _):
        dov[pl.ds(i * 16, 16)] = zeros
        div[pl.ds(i * 16, 16)] = zeros
        return 0
    lax.fori_loop(0, NPAD // 16, zbody, 0)

    pltpu.sync_copy(src_hbm.at[wid], idxv)

    def sbody(i, _):
        v = idxv[pl.ds(i * 16, 16)]
        plsc.addupdate_scatter(dov, [v], ones)
        return 0
    lax.fori_loop(0, EPW // 16, sbody, 0)

    pltpu.sync_copy(dst_hbm.at[wid], idxv)

    def dbody(i, _):
        v = idxv[pl.ds(i * 16, 16)]
        plsc.addupdate_scatter(div, [v], ones)
        return 0
    lax.fori_loop(0, EPW // 16, dbody, 0)

    pltpu.sync_copy(dov, dout_hbm.at[wid])
    pltpu.sync_copy(div, din_hbm.at[wid])


@functools.partial(
    pl.kernel,
    out_type=jax.ShapeDtypeStruct((NW, NPAD), F32),
    mesh=_mesh,
    compiler_params=_sc_params,
    scratch_types=[pltpu.VMEM((EPW,), jnp.int32),
                   pltpu.VMEM((EPW,), jnp.int32),
                   pltpu.VMEM((NPAD,), F32),
                   pltpu.VMEM((NPAD,), F32)],
)
def _sc_edge_weights(src_hbm, dst_hbm, nd_hbm, cpart_hbm, sidx, didx, ndv, cv):
    cid = lax.axis_index("c")
    sid = lax.axis_index("s")
    wid = sid * NC + cid
    zeros = jnp.zeros((16,), F32)

    def zbody(i, _):
        cv[pl.ds(i * 16, 16)] = zeros
        return 0
    lax.fori_loop(0, NPAD // 16, zbody, 0)

    pltpu.sync_copy(src_hbm.at[wid], sidx)
    pltpu.sync_copy(dst_hbm.at[wid], didx)
    pltpu.sync_copy(nd_hbm, ndv)

    def cbody(i, _):
        s16 = sidx[pl.ds(i * 16, 16)]
        d16 = didx[pl.ds(i * 16, 16)]
        nv = plsc.load_gather(ndv, [d16])
        plsc.addupdate_scatter(cv, [s16], nv)
        return 0
    lax.fori_loop(0, EPW // 16, cbody, 0)

    pltpu.sync_copy(cv, cpart_hbm.at[wid])


@functools.partial(
    pl.kernel,
    out_type=jax.ShapeDtypeStruct((NC, NPAD, D), F32),
    mesh=_mesh,
    compiler_params=_sc_params,
    scratch_types=[pltpu.VMEM((4, 2, K), jnp.int32),
                   pltpu.VMEM((K, D), F32),
                   pltpu.VMEM((K, D), F32),
                   pltpu.VMEM_SHARED((NPAD, D), F32),
                   pltpu.SemaphoreType.DMA,
                   pltpu.SemaphoreType.DMA,
                   pltpu.SemaphoreType.DMA,
                   pltpu.SemaphoreType.DMA,
                   pltpu.SemaphoreType.DMA,
                   pltpu.SemaphoreType.DMA],
)
def _sc_propagate(yn_hbm, e3_hbm, tok_hbm, acc_hbm,
                  ibuf, rows0, rows1, acc,
                  isem0, isem1, isem2, isem3, gsem0, gsem1):
    del tok_hbm
    cid = lax.axis_index("c")
    sid = lax.axis_index("s")
    wid = sid * NC + cid

    def ibody(i, _):
        r0 = sid * RPT + i * K
        pltpu.sync_copy(yn_hbm.at[pl.ds(r0, K)], rows0)
        pltpu.sync_copy(rows0, acc.at[pl.ds(r0, K)])
        return 0
    lax.fori_loop(0, RPT // K, ibody, 0)

    plsc.subcore_barrier()

    isems = (isem0, isem1, isem2, isem3)
    gsems = (gsem0, gsem1)
    rowbufs = (rows0, rows1)

    def fetch(j, slot):
        pltpu.async_copy(e3_hbm.at[wid, j], ibuf.at[slot], isems[slot])

    def wait_fetch(slot):
        pltpu.make_async_copy(e3_hbm.at[wid, 0], ibuf.at[slot],
                              isems[slot]).wait()

    def gather(slot):
        return pltpu.async_copy(yn_hbm.at[ibuf.at[slot, 0]],
                                rowbufs[slot % 2], gsems[slot % 2])

    def scatter(slot):
        pltpu.sync_copy(rowbufs[slot % 2], acc.at[ibuf.at[slot, 1]],
                        add=True)

    for s in range(4):
        fetch(s, s)

    def ebody(q, _):
        j = q * 4
        wait_fetch(0)
        g0 = gather(0)
        wait_fetch(1)
        g1 = gather(1)
        g0.wait()
        scatter(0)

        @pl.when(q < CH // 4 - 1)
        def _():
            fetch(j + 4, 0)
        wait_fetch(2)
        g2 = gather(2)
        g1.wait()
        scatter(1)

        @pl.when(q < CH // 4 - 1)
        def _():
            fetch(j + 5, 1)
        wait_fetch(3)
        g3 = gather(3)
        g2.wait()
        scatter(2)

        @pl.when(q < CH // 4 - 1)
        def _():
            fetch(j + 6, 2)
        g3.wait()
        scatter(3)

        @pl.when(q < CH // 4 - 1)
        def _():
            fetch(j + 7, 3)
        return 0
    lax.fori_loop(0, CH // 4, ebody, 0)

    plsc.subcore_barrier()

    def obody(i, _):
        r0 = sid * RPT + i * K
        pltpu.sync_copy(acc.at[pl.ds(r0, K)], rows0)
        pltpu.sync_copy(rows0, acc_hbm.at[cid, pl.ds(r0, K)])
        return 0
    lax.fori_loop(0, RPT // K, obody, 0)


def _tc_norms_l1(dout_ref, din_ref, x_ref, w1_ref, ns_ref, nd_ref, yn_ref):
    deg_out = jnp.sum(dout_ref[...], axis=0) + 1.0
    deg_in = jnp.sum(din_ref[...], axis=0) + 1.0
    ns = lax.rsqrt(deg_out)
    nd = lax.rsqrt(deg_in)
    ns_ref[...] = ns
    nd_ref[...] = nd
    y = jnp.dot(x_ref[...], w1_ref[...], preferred_element_type=F32)
    yn_ref[...] = y * ns[:, None]


def _tc_mid(acc_ref, yn_ref, nd_ref, b1_ref, w2_ref, ns_ref, cp_ref,
            mask_ref, y2n_ref, w_ref):
    agg = (acc_ref[0] + acc_ref[1] - yn_ref[...]) * nd_ref[...][:, None]
    h1 = jnp.maximum(agg + b1_ref[...][None, :], 0.0)
    y2 = jnp.dot(h1, w2_ref[...], preferred_element_type=F32)
    y2n_ref[...] = y2 * ns_ref[...][:, None]
    cfull = jnp.sum(cp_ref[...], axis=0) + nd_ref[...]
    w_ref[...] = mask_ref[...] * ns_ref[...] * cfull * (1.0 / N)


def _tc_final(acc_ref, yn_ref, nd_ref, b2_ref, wvec_ref, w4_ref, b4_ref,
              out_ref):
    agg = (acc_ref[0] + acc_ref[1] - yn_ref[...]) * nd_ref[...][:, None]
    h2 = jnp.maximum(agg + b2_ref[...][None, :], 0.0)
    s = jnp.dot(wvec_ref[...][None, :], h2, preferred_element_type=F32)
    out_ref[...] = (jnp.dot(s, w4_ref[...], preferred_element_type=F32)
                    + b4_ref[...][None, :])


def kernel(x, edge_index, W1, b1, W2, b2, W4, b4):
    src = edge_index[0].astype(jnp.int32)
    dst = edge_index[1].astype(jnp.int32)
    pad = jnp.full((EPAD - src.shape[0],), N, jnp.int32)
    src3 = jnp.concatenate([src, pad]).reshape(NW, CH, K)
    dst3 = jnp.concatenate([dst, pad]).reshape(NW, CH, K)
    e3 = jnp.stack([src3, dst3], axis=2)
    src2 = src3.reshape(NW, EPW)
    dst2 = dst3.reshape(NW, EPW)
    xpad = jnp.pad(x, ((0, NPAD - N), (0, 0)))
    mask = (jnp.arange(NPAD) < N).astype(F32)

    dout, din = _sc_degrees(src2, dst2)

    ns, nd, y1n = pl.pallas_call(
        _tc_norms_l1,
        out_shape=[jax.ShapeDtypeStruct((NPAD,), F32),
                   jax.ShapeDtypeStruct((NPAD,), F32),
                   jax.ShapeDtypeStruct((NPAD, D), F32)],
    )(dout, din, xpad, W1)

    cpart = _sc_edge_weights(src2, dst2, nd)
    acc1 = _sc_propagate(y1n, e3, cpart[:1, :8])

    y2n, wvec = pl.pallas_call(
        _tc_mid,
        out_shape=[jax.ShapeDtypeStruct((NPAD, D), F32),
                   jax.ShapeDtypeStruct((NPAD,), F32)],
    )(acc1, y1n, nd, b1, W2, ns, cpart, mask)

    acc2 = _sc_propagate(y2n, e3, y2n[:1, :8])

    out = pl.pallas_call(
        _tc_final,
        out_shape=jax.ShapeDtypeStruct((1, D), F32),
    )(acc2, y2n, nd, b2, wvec, W4, b4)

    return out.reshape(D)


# speedup vs baseline: 7.0587x; 1.0431x over previous
import functools

import jax
import jax.numpy as jnp
from jax import lax
from jax.experimental import pallas as pl
from jax.experimental.pallas import tpu as pltpu
from jax.experimental.pallas import tpu_sc as plsc

N = 10000
D = 128
NC = 2
NS = 16
NW = NC * NS
K = 128
CH = 80
EPW = CH * K
EPAD = NW * EPW
NPAD = 10240
RPT = NPAD // NS
F32 = jnp.float32

_mesh = plsc.VectorSubcoreMesh(core_axis_name="c", subcore_axis_name="s")
_sc_params = pltpu.CompilerParams(needs_layout_passes=False)


@functools.partial(
    pl.kernel,
    out_type=[jax.ShapeDtypeStruct((NW, NPAD), F32),
              jax.ShapeDtypeStruct((NW, NPAD), F32)],
    mesh=_mesh,
    compiler_params=_sc_params,
    scratch_types=[pltpu.VMEM((EPW,), jnp.int32),
                   pltpu.VMEM((NPAD,), F32),
                   pltpu.VMEM((NPAD,), F32)],
)
def _sc_degrees(src_hbm, dst_hbm, dout_hbm, din_hbm, idxv, dov, div):
    cid = lax.axis_index("c")
    sid = lax.axis_index("s")
    wid = sid * NC + cid
    zeros = jnp.zeros((16,), F32)
    ones = jnp.ones((16,), F32)

    def zbody(i, _):
        dov[pl.ds(i * 16, 16)] = zeros
        div[pl.ds(i * 16, 16)] = zeros
        return 0
    lax.fori_loop(0, NPAD // 16, zbody, 0)

    pltpu.sync_copy(src_hbm.at[wid], idxv)

    def sbody(i, _):
        v = idxv[pl.ds(i * 16, 16)]
        plsc.addupdate_scatter(dov, [v], ones)
        return 0
    lax.fori_loop(0, EPW // 16, sbody, 0)

    pltpu.sync_copy(dst_hbm.at[wid], idxv)

    def dbody(i, _):
        v = idxv[pl.ds(i * 16, 16)]
        plsc.addupdate_scatter(div, [v], ones)
        return 0
    lax.fori_loop(0, EPW // 16, dbody, 0)

    pltpu.sync_copy(dov, dout_hbm.at[wid])
    pltpu.sync_copy(div, din_hbm.at[wid])


@functools.partial(
    pl.kernel,
    out_type=jax.ShapeDtypeStruct((NW, NPAD), F32),
    mesh=_mesh,
    compiler_params=_sc_params,
    scratch_types=[pltpu.VMEM((EPW,), jnp.int32),
                   pltpu.VMEM((EPW,), jnp.int32),
                   pltpu.VMEM((NPAD,), F32),
                   pltpu.VMEM((NPAD,), F32)],
)
def _sc_edge_weights(src_hbm, dst_hbm, nd_hbm, cpart_hbm, sidx, didx, ndv, cv):
    cid = lax.axis_index("c")
    sid = lax.axis_index("s")
    wid = sid * NC + cid
    zeros = jnp.zeros((16,), F32)

    def zbody(i, _):
        cv[pl.ds(i * 16, 16)] = zeros
        return 0
    lax.fori_loop(0, NPAD // 16, zbody, 0)

    pltpu.sync_copy(src_hbm.at[wid], sidx)
    pltpu.sync_copy(dst_hbm.at[wid], didx)
    pltpu.sync_copy(nd_hbm, ndv)

    def cbody(i, _):
        s16 = sidx[pl.ds(i * 16, 16)]
        d16 = didx[pl.ds(i * 16, 16)]
        nv = plsc.load_gather(ndv, [d16])
        plsc.addupdate_scatter(cv, [s16], nv)
        return 0
    lax.fori_loop(0, EPW // 16, cbody, 0)

    pltpu.sync_copy(cv, cpart_hbm.at[wid])


@functools.partial(
    pl.kernel,
    out_type=jax.ShapeDtypeStruct((NC, NPAD, D), F32),
    mesh=_mesh,
    compiler_params=_sc_params,
    scratch_types=[pltpu.VMEM((4, 2, K), jnp.int32),
                   pltpu.VMEM((K, D), F32),
                   pltpu.VMEM((K, D), F32),
                   pltpu.VMEM_SHARED((NPAD, D), F32),
                   pltpu.SemaphoreType.DMA,
                   pltpu.SemaphoreType.DMA,
                   pltpu.SemaphoreType.DMA,
                   pltpu.SemaphoreType.DMA,
                   pltpu.SemaphoreType.DMA,
                   pltpu.SemaphoreType.DMA,
                   pltpu.SemaphoreType.DMA,
                   pltpu.SemaphoreType.DMA],
)
def _sc_propagate(yn_hbm, e3_hbm, tok_hbm, acc_hbm,
                  ibuf, rows0, rows1, acc,
                  isem0, isem1, isem2, isem3, gsem0, gsem1, ssem0, ssem1):
    del tok_hbm
    cid = lax.axis_index("c")
    sid = lax.axis_index("s")
    wid = sid * NC + cid

    def ibody(i, _):
        r0 = sid * RPT + i * K
        pltpu.sync_copy(yn_hbm.at[pl.ds(r0, K)], rows0)
        pltpu.sync_copy(rows0, acc.at[pl.ds(r0, K)])
        return 0
    lax.fori_loop(0, RPT // K, ibody, 0)

    plsc.subcore_barrier()

    isems = (isem0, isem1, isem2, isem3)
    gsems = (gsem0, gsem1)
    ssems = (ssem0, ssem1)
    rowbufs = (rows0, rows1)

    def fetch(j, slot):
        pltpu.async_copy(e3_hbm.at[wid, j], ibuf.at[slot], isems[slot])

    def wait_fetch(slot):
        pltpu.make_async_copy(e3_hbm.at[wid, 0], ibuf.at[slot],
                              isems[slot]).wait()

    def gather(slot):
        return pltpu.async_copy(yn_hbm.at[ibuf.at[slot, 0]],
                                rowbufs[slot % 2], gsems[slot % 2])

    def scatter(slot):
        return pltpu.async_copy(rowbufs[slot % 2], acc.at[ibuf.at[slot, 1]],
                                ssems[slot % 2], add=True)

    for s in range(4):
        fetch(s, s)

    def ebody(q, _):
        j = q * 4
        wait_fetch(0)
        g0 = gather(0)
        wait_fetch(1)
        g1 = gather(1)
        g0.wait()
        s0 = scatter(0)
        g1.wait()
        s1 = scatter(1)

        s0.wait()

        @pl.when(q < CH // 4 - 1)
        def _():
            fetch(j + 4, 0)
        wait_fetch(2)
        g2 = gather(2)
        s1.wait()

        @pl.when(q < CH // 4 - 1)
        def _():
            fetch(j + 5, 1)
        wait_fetch(3)
        g3 = gather(3)
        g2.wait()
        s2 = scatter(2)
        g3.wait()
        s3 = scatter(3)
        s2.wait()

        @pl.when(q < CH // 4 - 1)
        def _():
            fetch(j + 6, 2)
        s3.wait()

        @pl.when(q < CH // 4 - 1)
        def _():
            fetch(j + 7, 3)
        return 0
    lax.fori_loop(0, CH // 4, ebody, 0)

    plsc.subcore_barrier()

    def obody(i, _):
        r0 = sid * RPT + i * K
        pltpu.sync_copy(acc.at[pl.ds(r0, K)], rows0)
        pltpu.sync_copy(rows0, acc_hbm.at[cid, pl.ds(r0, K)])
        return 0
    lax.fori_loop(0, RPT // K, obody, 0)


def _tc_norms_l1(dout_ref, din_ref, x_ref, w1_ref, ns_ref, nd_ref, yn_ref):
    deg_out = jnp.sum(dout_ref[...], axis=0) + 1.0
    deg_in = jnp.sum(din_ref[...], axis=0) + 1.0
    ns = lax.rsqrt(deg_out)
    nd = lax.rsqrt(deg_in)
    ns_ref[...] = ns
    nd_ref[...] = nd
    y = jnp.dot(x_ref[...], w1_ref[...], preferred_element_type=F32)
    yn_ref[...] = y * ns[:, None]


def _tc_mid(acc_ref, yn_ref, nd_ref, b1_ref, w2_ref, ns_ref, cp_ref,
            mask_ref, y2n_ref, w_ref):
    agg = (acc_ref[0] + acc_ref[1] - yn_ref[...]) * nd_ref[...][:, None]
    h1 = jnp.maximum(agg + b1_ref[...][None, :], 0.0)
    y2 = jnp.dot(h1, w2_ref[...], preferred_element_type=F32)
    y2n_ref[...] = y2 * ns_ref[...][:, None]
    cfull = jnp.sum(cp_ref[...], axis=0) + nd_ref[...]
    w_ref[...] = mask_ref[...] * ns_ref[...] * cfull * (1.0 / N)


def _tc_final(acc_ref, yn_ref, nd_ref, b2_ref, wvec_ref, w4_ref, b4_ref,
              out_ref):
    agg = (acc_ref[0] + acc_ref[1] - yn_ref[...]) * nd_ref[...][:, None]
    h2 = jnp.maximum(agg + b2_ref[...][None, :], 0.0)
    s = jnp.dot(wvec_ref[...][None, :], h2, preferred_element_type=F32)
    out_ref[...] = (jnp.dot(s, w4_ref[...], preferred_element_type=F32)
                    + b4_ref[...][None, :])


def kernel(x, edge_index, W1, b1, W2, b2, W4, b4):
    src = edge_index[0].astype(jnp.int32)
    dst = edge_index[1].astype(jnp.int32)
    pad = jnp.full((EPAD - src.shape[0],), N, jnp.int32)
    src3 = jnp.concatenate([src, pad]).reshape(NW, CH, K)
    dst3 = jnp.concatenate([dst, pad]).reshape(NW, CH, K)
    e3 = jnp.stack([src3, dst3], axis=2)
    src2 = src3.reshape(NW, EPW)
    dst2 = dst3.reshape(NW, EPW)
    xpad = jnp.pad(x, ((0, NPAD - N), (0, 0)))
    mask = (jnp.arange(NPAD) < N).astype(F32)

    dout, din = _sc_degrees(src2, dst2)

    ns, nd, y1n = pl.pallas_call(
        _tc_norms_l1,
        out_shape=[jax.ShapeDtypeStruct((NPAD,), F32),
                   jax.ShapeDtypeStruct((NPAD,), F32),
                   jax.ShapeDtypeStruct((NPAD, D), F32)],
    )(dout, din, xpad, W1)

    cpart = _sc_edge_weights(src2, dst2, nd)
    acc1 = _sc_propagate(y1n, e3, cpart[:1, :8])

    y2n, wvec = pl.pallas_call(
        _tc_mid,
        out_shape=[jax.ShapeDtypeStruct((NPAD, D), F32),
                   jax.ShapeDtypeStruct((NPAD,), F32)],
    )(acc1, y1n, nd, b1, W2, ns, cpart, mask)

    acc2 = _sc_propagate(y2n, e3, y2n[:1, :8])

    out = pl.pallas_call(
        _tc_final,
        out_shape=jax.ShapeDtypeStruct((1, D), F32),
    )(acc2, y2n, nd, b2, wvec, W4, b4)

    return out.reshape(D)


# speedup vs baseline: 8.1075x; 1.1486x over previous
import functools

import jax
import jax.numpy as jnp
from jax import lax
from jax.experimental import pallas as pl
from jax.experimental.pallas import tpu as pltpu
from jax.experimental.pallas import tpu_sc as plsc

N = 10000
D = 128
NC = 2
NS = 16
NW = NC * NS
K = 32
CH = 320
EPW = CH * K
CHP = 640
EPT = CHP * K
EPAD = NW * EPW
NPAD = 10240
NH = NPAD // NC
YR = 10016
SRT = 640
ART = NH // NS
F32 = jnp.float32

_mesh = plsc.VectorSubcoreMesh(core_axis_name="c", subcore_axis_name="s")
_sc_params = pltpu.CompilerParams(needs_layout_passes=False)


def _copy_rows(src_at, dst_at, total, buf):
    nfull = total // K
    tail = total - nfull * K

    def body(i, _):
        pltpu.sync_copy(src_at(i * K, K), buf.at[pl.ds(0, K)])
        pltpu.sync_copy(buf.at[pl.ds(0, K)], dst_at(i * K, K))
        return 0
    lax.fori_loop(0, nfull, body, 0)
    if tail:
        pltpu.sync_copy(src_at(nfull * K, tail), buf.at[pl.ds(0, tail)])
        pltpu.sync_copy(buf.at[pl.ds(0, tail)], dst_at(nfull * K, tail))


@functools.partial(
    pl.kernel,
    out_type=[jax.ShapeDtypeStruct((NW, NPAD), F32),
              jax.ShapeDtypeStruct((NW, NPAD), F32)],
    mesh=_mesh,
    compiler_params=_sc_params,
    scratch_types=[pltpu.VMEM((EPW,), jnp.int32),
                   pltpu.VMEM((NPAD,), F32),
                   pltpu.VMEM((NPAD,), F32)],
)
def _sc_degrees(src_hbm, dst_hbm, dout_hbm, din_hbm, idxv, dov, div):
    cid = lax.axis_index("c")
    sid = lax.axis_index("s")
    wid = sid * NC + cid
    zeros = jnp.zeros((16,), F32)
    ones = jnp.ones((16,), F32)

    def zbody(i, _):
        dov[pl.ds(i * 16, 16)] = zeros
        div[pl.ds(i * 16, 16)] = zeros
        return 0
    lax.fori_loop(0, NPAD // 16, zbody, 0)

    pltpu.sync_copy(src_hbm.at[wid], idxv)

    def sbody(i, _):
        v = idxv[pl.ds(i * 16, 16)]
        plsc.addupdate_scatter(dov, [v], ones)
        return 0
    lax.fori_loop(0, EPW // 16, sbody, 0)

    pltpu.sync_copy(dst_hbm.at[wid], idxv)

    def dbody(i, _):
        v = idxv[pl.ds(i * 16, 16)]
        plsc.addupdate_scatter(div, [v], ones)
        return 0
    lax.fori_loop(0, EPW // 16, dbody, 0)

    pltpu.sync_copy(dov, dout_hbm.at[wid])
    pltpu.sync_copy(div, din_hbm.at[wid])


@functools.partial(
    pl.kernel,
    out_type=jax.ShapeDtypeStruct((NW, NPAD), F32),
    mesh=_mesh,
    compiler_params=_sc_params,
    scratch_types=[pltpu.VMEM((EPW,), jnp.int32),
                   pltpu.VMEM((EPW,), jnp.int32),
                   pltpu.VMEM((NPAD,), F32),
                   pltpu.VMEM((NPAD,), F32)],
)
def _sc_edge_weights(src_hbm, dst_hbm, nd_hbm, cpart_hbm, sidx, didx, ndv, cv):
    cid = lax.axis_index("c")
    sid = lax.axis_index("s")
    wid = sid * NC + cid
    zeros = jnp.zeros((16,), F32)

    def zbody(i, _):
        cv[pl.ds(i * 16, 16)] = zeros
        return 0
    lax.fori_loop(0, NPAD // 16, zbody, 0)

    pltpu.sync_copy(src_hbm.at[wid], sidx)
    pltpu.sync_copy(dst_hbm.at[wid], didx)
    pltpu.sync_copy(nd_hbm, ndv)

    def cbody(i, _):
        s16 = sidx[pl.ds(i * 16, 16)]
        d16 = didx[pl.ds(i * 16, 16)]
        nv = plsc.load_gather(ndv, [d16])
        plsc.addupdate_scatter(cv, [s16], nv)
        return 0
    lax.fori_loop(0, EPW // 16, cbody, 0)

    pltpu.sync_copy(cv, cpart_hbm.at[wid])


@functools.partial(
    pl.kernel,
    out_type=jax.ShapeDtypeStruct((NC, NH, D), F32),
    mesh=_mesh,
    compiler_params=_sc_params,
    scratch_types=[pltpu.VMEM((4, 2, K), jnp.int32),
                   pltpu.VMEM((4, K), jnp.int32),
                   pltpu.VMEM((K, D), F32),
                   pltpu.VMEM((K, D), F32),
                   pltpu.VMEM_SHARED((YR, D), F32),
                   pltpu.VMEM_SHARED((NH + 8, D), F32),
                   pltpu.SemaphoreType.DMA,
                   pltpu.SemaphoreType.DMA,
                   pltpu.SemaphoreType.DMA,
                   pltpu.SemaphoreType.DMA,
                   pltpu.SemaphoreType.DMA,
                   pltpu.SemaphoreType.DMA,
                   pltpu.SemaphoreType.DMA,
                   pltpu.SemaphoreType.DMA],
)
def _sc_propagate(yn_hbm, e3_hbm, tok_hbm, acc_hbm,
                  ibuf, dstloc, rows0, rows1, ynsp, acch,
                  isem0, isem1, isem2, isem3, gsem0, gsem1, ssem0, ssem1):
    del tok_hbm
    cid = lax.axis_index("c")
    sid = lax.axis_index("s")
    base = cid * NH

    @pl.when(sid < NS - 1)
    def _():
        _copy_rows(lambda r, n: yn_hbm.at[pl.ds(sid * SRT + r, n)],
                   lambda r, n: ynsp.at[pl.ds(sid * SRT + r, n)], SRT, rows0)

    @pl.when(sid == NS - 1)
    def _():
        _copy_rows(lambda r, n: yn_hbm.at[pl.ds((NS - 1) * SRT + r, n)],
                   lambda r, n: ynsp.at[pl.ds((NS - 1) * SRT + r, n)],
                   YR - (NS - 1) * SRT, rows0)

    _copy_rows(lambda r, n: yn_hbm.at[pl.ds(base + sid * ART + r, n)],
               lambda r, n: acch.at[pl.ds(sid * ART + r, n)], ART, rows0)

    plsc.subcore_barrier()

    isems = (isem0, isem1, isem2, isem3)
    gsems = (gsem0, gsem1)
    ssems = (ssem0, ssem1)
    rowbufs = (rows0, rows1)

    def fetch(j, slot):
        pltpu.async_copy(e3_hbm.at[sid, j], ibuf.at[slot], isems[slot])

    def wait_fetch(slot):
        pltpu.make_async_copy(e3_hbm.at[sid, 0], ibuf.at[slot],
                              isems[slot]).wait()

    def remap(slot):
        for v in range(K // 16):
            d16 = ibuf[slot, 1, pl.ds(v * 16, 16)]
            loc = d16 - base
            inb = (loc >= 0) & (loc < NH)
            dstloc[slot, pl.ds(v * 16, 16)] = jnp.where(inb, loc, NH)

    def gather(slot):
        return pltpu.async_copy(ynsp.at[ibuf.at[slot, 0]],
                                rowbufs[slot % 2], gsems[slot % 2])

    def scatter(slot):
        return pltpu.async_copy(rowbufs[slot % 2], acch.at[dstloc.at[slot]],
                                ssems[slot % 2], add=True)

    for s in range(4):
        fetch(s, s)

    def ebody(q, _):
        j = q * 4
        wait_fetch(0)
        remap(0)
        g0 = gather(0)
        wait_fetch(1)
        remap(1)
        g1 = gather(1)
        g0.wait()
        s0 = scatter(0)
        g1.wait()
        s1 = scatter(1)

        s0.wait()

        @pl.when(q < CHP // 4 - 1)
        def _():
            fetch(j + 4, 0)
        wait_fetch(2)
        remap(2)
        g2 = gather(2)
        s1.wait()

        @pl.when(q < CHP // 4 - 1)
        def _():
            fetch(j + 5, 1)
        wait_fetch(3)
        remap(3)
        g3 = gather(3)
        g2.wait()
        s2 = scatter(2)
        g3.wait()
        s3 = scatter(3)
        s2.wait()

        @pl.when(q < CHP // 4 - 1)
        def _():
            fetch(j + 6, 2)
        s3.wait()

        @pl.when(q < CHP // 4 - 1)
        def _():
            fetch(j + 7, 3)
        return 0
    lax.fori_loop(0, CHP // 4, ebody, 0)

    plsc.subcore_barrier()

    _copy_rows(lambda r, n: acch.at[pl.ds(sid * ART + r, n)],
               lambda r, n: acc_hbm.at[cid, pl.ds(sid * ART + r, n)],
               ART, rows0)


def _tc_norms_l1(dout_ref, din_ref, x_ref, w1_ref, ns_ref, nd_ref, yn_ref):
    deg_out = jnp.sum(dout_ref[...], axis=0) + 1.0
    deg_in = jnp.sum(din_ref[...], axis=0) + 1.0
    ns = lax.rsqrt(deg_out)
    nd = lax.rsqrt(deg_in)
    ns_ref[...] = ns
    nd_ref[...] = nd
    y = jnp.dot(x_ref[...], w1_ref[...], preferred_element_type=F32)
    yn_ref[...] = y * ns[:, None]


def _tc_mid(acc_ref, nd_ref, b1_ref, w2_ref, ns_ref, cp_ref,
            mask_ref, y2n_ref, w_ref):
    agg = jnp.concatenate([acc_ref[0], acc_ref[1]], axis=0)
    agg = agg * nd_ref[...][:, None]
    h1 = jnp.maximum(agg + b1_ref[...][None, :], 0.0)
    y2 = jnp.dot(h1, w2_ref[...], preferred_element_type=F32)
    y2n_ref[...] = y2 * ns_ref[...][:, None]
    cfull = jnp.sum(cp_ref[...], axis=0) + nd_ref[...]
    w_ref[...] = mask_ref[...] * ns_ref[...] * cfull * (1.0 / N)


def _tc_final(acc_ref, nd_ref, b2_ref, wvec_ref, w4_ref, b4_ref, out_ref):
    agg = jnp.concatenate([acc_ref[0], acc_ref[1]], axis=0)
    agg = agg * nd_ref[...][:, None]
    h2 = jnp.maximum(agg + b2_ref[...][None, :], 0.0)
    s = jnp.dot(wvec_ref[...][None, :], h2, preferred_element_type=F32)
    out_ref[...] = (jnp.dot(s, w4_ref[...], preferred_element_type=F32)
                    + b4_ref[...][None, :])


def kernel(x, edge_index, W1, b1, W2, b2, W4, b4):
    src = edge_index[0].astype(jnp.int32)
    dst = edge_index[1].astype(jnp.int32)
    pad = jnp.full((EPAD - src.shape[0],), N, jnp.int32)
    srcp = jnp.concatenate([src, pad])
    dstp = jnp.concatenate([dst, pad])
    e3 = jnp.stack([srcp.reshape(NS, CHP, K),
                    dstp.reshape(NS, CHP, K)], axis=2)
    src2 = srcp.reshape(NW, EPW)
    dst2 = dstp.reshape(NW, EPW)
    xpad = jnp.pad(x, ((0, NPAD - N), (0, 0)))
    mask = (jnp.arange(NPAD) < N).astype(F32)

    dout, din = _sc_degrees(src2, dst2)

    ns, nd, y1n = pl.pallas_call(
        _tc_norms_l1,
        out_shape=[jax.ShapeDtypeStruct((NPAD,), F32),
                   jax.ShapeDtypeStruct((NPAD,), F32),
                   jax.ShapeDtypeStruct((NPAD, D), F32)],
    )(dout, din, xpad, W1)

    cpart = _sc_edge_weights(src2, dst2, nd)
    acc1 = _sc_propagate(y1n, e3, cpart[:1, :8])

    y2n, wvec = pl.pallas_call(
        _tc_mid,
        out_shape=[jax.ShapeDtypeStruct((NPAD, D), F32),
                   jax.ShapeDtypeStruct((NPAD,), F32)],
    )(acc1, nd, b1, W2, ns, cpart, mask)

    acc2 = _sc_propagate(y2n, e3, y2n[:1, :8])

    out = pl.pallas_call(
        _tc_final,
        out_shape=jax.ShapeDtypeStruct((1, D), F32),
    )(acc2, nd, b2, wvec, W4, b4)

    return out.reshape(D)
